# baseline (speedup 1.0000x reference)
import jax
import jax.numpy as jnp
from jax.experimental import pallas as pl
from jax.experimental.pallas import tpu as pltpu

_EPS = 1e-12
_LOG2E = 1.4426950408889634


def _body(xq_ref, fw_ref, w1_ref, b1_ref, w2_ref, b2t_ref, out_ref,
          mn_ref, vp_ref):
    c = xq_ref.shape[0]

    @pl.when(pl.program_id(0) == 0)
    def _prep():
        fw = fw_ref[...]
        m = fw[:, :c]
        n1 = jnp.sqrt(jnp.sum(m * m, axis=1, keepdims=True))
        mn_ref[...] = (m / jnp.maximum(n1, _EPS)).astype(jnp.bfloat16)
        n2 = jnp.sqrt(jnp.sum(fw * fw, axis=1, keepdims=True))
        fwn = fw / jnp.maximum(n2, _EPS)
        vp_ref[...] = jnp.dot(
            fwn, w1_ref[...], preferred_element_type=jnp.float32
        ).astype(jnp.bfloat16)

    xqt = xq_ref[...]
    nrm = jnp.sqrt(jnp.sum(xqt * xqt, axis=0, keepdims=True))
    xnt = (xqt * (_LOG2E / jnp.maximum(nrm, _EPS))).astype(jnp.bfloat16)
    s = jax.lax.dot_general(
        xnt, mn_ref[...], (((0,), (1,)), ((), ())),
        preferred_element_type=jnp.float32)
    e = jnp.exp2(s.astype(jnp.bfloat16))
    acc = jnp.dot(e, vp_ref[...],
                  preferred_element_type=jnp.float32)
    kk = e.shape[1]
    e2 = e[:, :kk // 2] + e[:, kk // 2:]
    e4 = e2[:, :kk // 4] + e2[:, kk // 4:]
    den = jnp.sum(e4.astype(jnp.float32), axis=1, keepdims=True)
    o = acc / den + b1_ref[...]
    h1 = 0.5 * o * (1.0 + jax.lax.erf(o * (2.0 ** -0.5)))
    out_ref[...] = (jax.lax.dot_general(
        w2_ref[...], h1, (((0,), (1,)), ((), ())),
        preferred_element_type=jnp.float32) + b2t_ref[...])


def kernel(x, feat_w, W1, b1, W2, b2):
    b, c, h, w = x.shape
    n = b * h * w
    k, c4 = feat_w.shape
    hdim = W2.shape[1]

    hw = h * w
    x2 = x.reshape(b * c, hw)
    b1_2d = b1.reshape(1, hdim)
    b2_col = b2.reshape(hdim, 1)

    nq = b
    out2d = pl.pallas_call(
        _body,
        grid=(nq,),
        in_specs=[
            pl.BlockSpec((c, hw), lambda i: (i, 0)),
            pl.BlockSpec((k, c4), lambda i: (0, 0)),
            pl.BlockSpec((c4, hdim), lambda i: (0, 0)),
            pl.BlockSpec((1, hdim), lambda i: (0, 0)),
            pl.BlockSpec((hdim, hdim), lambda i: (0, 0)),
            pl.BlockSpec((hdim, 1), lambda i: (0, 0)),
        ],
        out_specs=pl.BlockSpec((hdim, hw), lambda i: (i, 0)),
        out_shape=jax.ShapeDtypeStruct((b * hdim, hw), jnp.float32),
        scratch_shapes=[
            pltpu.VMEM((k, c), jnp.bfloat16),
            pltpu.VMEM((k, hdim), jnp.bfloat16),
        ],
        compiler_params=pltpu.CompilerParams(
            dimension_semantics=("arbitrary",)),
    )(x2, feat_w, W1, b1_2d, W2, b2_col)

    return out2d.reshape(b, hdim, h, w)
